# baseline (device time: 39627 ns/iter reference)
import jax
import jax.numpy as jnp
from jax import lax
from jax.experimental import pallas as pl
from jax.experimental.pallas import tpu as pltpu

N_DEV = 16
E_PER = 2
CAP_E = 32
A_LANES = 128

ORDER = sorted(range(1, N_DEV), key=lambda o: -min(o, N_DEV - o))


def kernel(x, assign, W1, W2):
    t, d = x.shape
    e, _, f = W1.shape
    n = N_DEV * E_PER * CAP_E

    xb = x.astype(jnp.bfloat16)
    ab = assign.reshape(t, 1)

    def body(x_ref, a_ref, w1_ref, w2_ref, out_ref,
             sb0, sb1, rv0, rv1, yl0, yl1, rb0, rb1,
             s1a, r1a, s1b, r1b, s2a, r2a, s2b, r2b, loc_sems):
        my = lax.axis_index("i")
        sb = (sb0, sb1)
        rv = (rv0, rv1)
        yl = (yl0, yl1)
        rb = (rb0, rb1)
        s1 = (s1a, s1b)
        r1 = (r1a, r1b)
        s2 = (s2a, s2b)
        r2 = (r2a, r2b)

        barrier = pltpu.get_barrier_semaphore()
        for o in range(1, N_DEV):
            q = lax.rem(my + o, N_DEV)
            pl.semaphore_signal(barrier, inc=1, device_id=(q,),
                                device_id_type=pl.DeviceIdType.MESH)

        a = a_ref[...]
        lane = lax.broadcasted_iota(jnp.int32, (t, A_LANES), 1)
        da = a - lane
        oh = jnp.maximum(1 - da * da, 0).astype(jnp.bfloat16)
        dtr = (lax.broadcasted_iota(jnp.int32, (t, t), 0)
               - lax.broadcasted_iota(jnp.int32, (t, t), 1))
        tril = jnp.clip(dtr, 0, 1).astype(jnp.bfloat16)
        rm = lax.dot_general(tril, oh, (((1,), (0,)), ((), ())),
                             preferred_element_type=jnp.float32)
        rank = jnp.sum(oh.astype(jnp.float32) * rm, axis=1, keepdims=True
                       ).astype(jnp.int32)
        slot = a * CAP_E + rank
        dsl = slot - lax.broadcasted_iota(jnp.int32, (t, n), 1)
        perm = jnp.maximum(1 - dsl * dsl, 0).astype(jnp.bfloat16)

        send = lax.dot_general(perm, x_ref[...], (((0,), (0,)), ((), ())),
                               preferred_element_type=jnp.float32)
        sendv = send.astype(jnp.bfloat16).reshape(N_DEV, E_PER, CAP_E, d)
        sb0[...] = sendv[:, 0]
        sb1[...] = sendv[:, 1]

        pl.semaphore_wait(barrier, N_DEV - 1)

        ph1 = ([], [])
        cps_in = []
        for ei in range(E_PER):
            cp = pltpu.make_async_copy(sb[ei].at[my], rv[ei].at[my],
                                       loc_sems.at[ei])
            cp.start()
            cps_in.append(cp)
            for o in ORDER:
                j = lax.rem(my + o, N_DEV)
                r = pltpu.make_async_remote_copy(
                    src_ref=sb[ei].at[j], dst_ref=rv[ei].at[my],
                    send_sem=s1[ei].at[o], recv_sem=r1[ei].at[o],
                    device_id=(j,), device_id_type=pl.DeviceIdType.MESH)
                r.start()
                ph1[ei].append(r)

        ph2 = ([], [])
        cps_out = []
        for ei in range(E_PER):
            cps_in[ei].wait()
            for r in ph1[ei]:
                r.wait()
            xe = rv[ei][...].reshape(N_DEV * CAP_E, d)
            w1e = w1_ref[ei].astype(jnp.bfloat16)
            h = lax.dot_general(xe, w1e, (((1,), (0,)), ((), ())),
                                preferred_element_type=jnp.float32)
            h = jnp.maximum(h, 0.0).astype(jnp.bfloat16)
            w2e = w2_ref[ei].astype(jnp.bfloat16)
            pe = lax.dot_general(h, w2e, (((1,), (0,)), ((), ())),
                                 preferred_element_type=jnp.float32)
            yl[ei][...] = pe.astype(jnp.bfloat16).reshape(N_DEV, CAP_E, d)

            cp = pltpu.make_async_copy(yl[ei].at[my], rb[ei].at[my],
                                       loc_sems.at[E_PER + ei])
            cp.start()
            cps_out.append(cp)
            for o in ORDER:
                j = lax.rem(my + o, N_DEV)
                r = pltpu.make_async_remote_copy(
                    src_ref=yl[ei].at[j], dst_ref=rb[ei].at[my],
                    send_sem=s2[ei].at[o], recv_sem=r2[ei].at[o],
                    device_id=(j,), device_id_type=pl.DeviceIdType.MESH)
                r.start()
                ph2[ei].append(r)

        for ei in range(E_PER):
            cps_out[ei].wait()
            for r in ph2[ei]:
                r.wait()

        y = jnp.concatenate([rb0[...], rb1[...]], axis=1).reshape(n, d)
        out_ref[...] = lax.dot_general(perm, y, (((1,), (0,)), ((), ())),
                                       preferred_element_type=jnp.float32)

    buf = pltpu.VMEM((N_DEV, CAP_E, d), jnp.bfloat16)
    sem16 = pltpu.SemaphoreType.DMA((N_DEV,))
    return pl.pallas_call(
        body,
        out_shape=jax.ShapeDtypeStruct((t, d), jnp.float32),
        in_specs=[pl.BlockSpec(memory_space=pltpu.VMEM)] * 4,
        out_specs=pl.BlockSpec(memory_space=pltpu.VMEM),
        scratch_shapes=[
            buf, buf,
            buf, buf,
            buf, buf,
            buf, buf,
            sem16, sem16,
            sem16, sem16,
            sem16, sem16,
            sem16, sem16,
            pltpu.SemaphoreType.DMA((2 * E_PER,)),
        ],
        compiler_params=pltpu.CompilerParams(collective_id=0),
    )(xb, ab, W1, W2)


# device time: 38964 ns/iter; 1.0170x vs baseline; 1.0170x over previous
import jax
import jax.numpy as jnp
from jax import lax
from jax.experimental import pallas as pl
from jax.experimental.pallas import tpu as pltpu

N_DEV = 16
E_PER = 2
CAP_E = 32
A_LANES = 128


def kernel(x, assign, W1, W2):
    t, d = x.shape
    e, _, f = W1.shape
    n = N_DEV * E_PER * CAP_E
    rows = E_PER * CAP_E

    xb = x.astype(jnp.bfloat16)
    ab = assign.reshape(t, 1)

    def body(x_ref, a_ref, w1_ref, w2_ref, out_ref,
             sendbuf, recv, yloc, retbuf,
             s1_sems, r1_sems, s2_sems, r2_sems, loc_sem):
        my = lax.axis_index("i")

        barrier = pltpu.get_barrier_semaphore()
        for o in range(1, N_DEV):
            q = lax.rem(my + o, N_DEV)
            pl.semaphore_signal(barrier, inc=1, device_id=(q,),
                                device_id_type=pl.DeviceIdType.MESH)

        a = a_ref[...]
        lane = lax.broadcasted_iota(jnp.int32, (t, A_LANES), 1)
        da = a - lane
        oh = jnp.maximum(1 - da * da, 0).astype(jnp.bfloat16)
        dtr = (lax.broadcasted_iota(jnp.int32, (t, t), 0)
               - lax.broadcasted_iota(jnp.int32, (t, t), 1))
        tril = jnp.clip(dtr, 0, 1).astype(jnp.bfloat16)
        rm = lax.dot_general(tril, oh, (((1,), (0,)), ((), ())),
                             preferred_element_type=jnp.float32)
        rank = jnp.sum(oh.astype(jnp.float32) * rm, axis=1, keepdims=True
                       ).astype(jnp.int32)
        slot = a * CAP_E + rank
        dsl = slot - lax.broadcasted_iota(jnp.int32, (t, n), 1)
        perm = jnp.maximum(1 - dsl * dsl, 0).astype(jnp.bfloat16)

        send = lax.dot_general(perm, x_ref[...], (((0,), (0,)), ((), ())),
                               preferred_element_type=jnp.float32)
        sendbuf[...] = send.astype(jnp.bfloat16).reshape(N_DEV, rows, d)

        pl.semaphore_wait(barrier, N_DEV - 1)

        cp_in = pltpu.make_async_copy(sendbuf.at[my], recv.at[my], loc_sem)
        cp_in.start()
        ph1 = []
        for o in range(1, N_DEV):
            j = lax.rem(my + o, N_DEV)
            r = pltpu.make_async_remote_copy(
                src_ref=sendbuf.at[j], dst_ref=recv.at[my],
                send_sem=s1_sems.at[o], recv_sem=r1_sems.at[o],
                device_id=(j,), device_id_type=pl.DeviceIdType.MESH)
            r.start()
            ph1.append(r)
        cp_in.wait()
        for r in ph1:
            r.wait()

        xarec = recv[...]
        pes = []
        for ei in range(e):
            xe = xarec[:, ei * CAP_E:(ei + 1) * CAP_E, :].reshape(
                N_DEV * CAP_E, d)
            w1e = w1_ref[ei].astype(jnp.bfloat16)
            h = lax.dot_general(xe, w1e, (((1,), (0,)), ((), ())),
                                preferred_element_type=jnp.float32)
            h = jnp.maximum(h, 0.0).astype(jnp.bfloat16)
            w2e = w2_ref[ei].astype(jnp.bfloat16)
            pe = lax.dot_general(h, w2e, (((1,), (0,)), ((), ())),
                                 preferred_element_type=jnp.float32)
            pes.append(pe.astype(jnp.bfloat16).reshape(N_DEV, CAP_E, d))
        yloc[...] = jnp.concatenate(pes, axis=1)

        cp_out = pltpu.make_async_copy(yloc.at[my], retbuf.at[my], loc_sem)
        cp_out.start()
        ph2 = []
        for o in range(1, N_DEV):
            j = lax.rem(my + o, N_DEV)
            r = pltpu.make_async_remote_copy(
                src_ref=yloc.at[j], dst_ref=retbuf.at[my],
                send_sem=s2_sems.at[o], recv_sem=r2_sems.at[o],
                device_id=(j,), device_id_type=pl.DeviceIdType.MESH)
            r.start()
            ph2.append(r)
        cp_out.wait()
        for r in ph2:
            r.wait()

        y = retbuf[...].reshape(n, d)
        out_ref[...] = lax.dot_general(perm, y, (((1,), (0,)), ((), ())),
                                       preferred_element_type=jnp.float32)

    return pl.pallas_call(
        body,
        out_shape=jax.ShapeDtypeStruct((t, d), jnp.float32),
        in_specs=[pl.BlockSpec(memory_space=pltpu.VMEM)] * 4,
        out_specs=pl.BlockSpec(memory_space=pltpu.VMEM),
        scratch_shapes=[
            pltpu.VMEM((N_DEV, rows, d), jnp.bfloat16),
            pltpu.VMEM((N_DEV, rows, d), jnp.bfloat16),
            pltpu.VMEM((N_DEV, rows, d), jnp.bfloat16),
            pltpu.VMEM((N_DEV, rows, d), jnp.bfloat16),
            pltpu.SemaphoreType.DMA((N_DEV,)),
            pltpu.SemaphoreType.DMA((N_DEV,)),
            pltpu.SemaphoreType.DMA((N_DEV,)),
            pltpu.SemaphoreType.DMA((N_DEV,)),
            pltpu.SemaphoreType.DMA,
        ],
        compiler_params=pltpu.CompilerParams(collective_id=0),
    )(xb, ab, W1, W2)
